# baseline (speedup 1.0000x reference)
import functools

import jax
import jax.numpy as jnp
from jax import lax
from jax.experimental import pallas as pl
from jax.experimental.pallas import tpu as pltpu
from jax.experimental.pallas import tpu_sc as plsc


def _make_sc_gather(Vp, D2, N):
    info = plsc.get_sparse_core_info()
    NC, NS, L = info.num_cores, info.num_subcores, info.num_lanes
    NW = NC * NS
    assert N % NW == 0
    per_w = N // NW
    CH = 128
    sizes = [CH] * (per_w // CH)
    if per_w % CH:
        sizes.append(per_w % CH)
    mesh = plsc.VectorSubcoreMesh(core_axis_name="c", subcore_axis_name="s")

    @functools.partial(
        pl.kernel,
        out_type=jax.ShapeDtypeStruct((N, D2), jnp.float32),
        mesh=mesh,
        scratch_types=[
            pltpu.VMEM((per_w,), jnp.int32),
            pltpu.VMEM((CH, D2), jnp.float32),
            pltpu.VMEM((CH, D2), jnp.float32),
            pltpu.SemaphoreType.DMA,
        ],
    )
    def gather_k(table_hbm, idx_hbm, out_hbm, idx_v, buf_a, buf_b, sem):
        wid = lax.axis_index("s") * NC + lax.axis_index("c")
        base = pl.multiple_of(wid * per_w, per_w)
        pltpu.sync_copy(idx_hbm.at[pl.ds(base, per_w)], idx_v)
        bufs = [buf_a, buf_b]

        def fire(g):
            sz = sizes[g]
            return pltpu.async_copy(
                table_hbm.at[idx_v.at[pl.ds(g * CH, sz)]],
                bufs[g % 2].at[pl.ds(0, sz)],
                sem,
            )

        cps = [None] * len(sizes)
        cps[0] = fire(0)
        for g in range(len(sizes)):
            if g + 1 < len(sizes):
                cps[g + 1] = fire(g + 1)
            cps[g].wait()
            sz = sizes[g]
            dst = pl.multiple_of(base + g * CH, 8)
            pltpu.sync_copy(
                bufs[g % 2].at[pl.ds(0, sz)], out_hbm.at[pl.ds(dst, sz)]
            )

    return gather_k


def _repack_tc(ET):
    Dd, V = ET.shape
    SB = 256
    NS = 16
    CB = SB * NS

    def body(perm_ref, in_ref, out_ref):
        perm = perm_ref[...]
        dn = (((1,), (1,)), ((), ()))
        for j in range(0, NS, 4):
            xb = jnp.concatenate(
                [in_ref[:, (j + k) * SB : (j + k + 1) * SB] for k in range(4)],
                axis=0,
            )
            m = lax.dot_general(perm, xb, dn,
                                preferred_element_type=jnp.float32)
            for k in range(4):
                out_ref[pl.ds((j + k) * (SB // 2), SB // 2), :] = (
                    jnp.concatenate(
                        [
                            m[: SB // 2, k * Dd : (k + 1) * Dd],
                            m[SB // 2 :, k * Dd : (k + 1) * Dd],
                        ],
                        axis=1,
                    )
                )

    ii = lax.broadcasted_iota(jnp.int32, (SB, SB), 0)
    jj = lax.broadcasted_iota(jnp.int32, (SB, SB), 1)
    perm_c = (jj == 2 * (ii & 127) + (ii >> 7)).astype(jnp.float32)

    return pl.pallas_call(
        body,
        grid=(pl.cdiv(V, CB),),
        in_specs=[
            pl.BlockSpec((SB, SB), lambda c: (0, 0)),
            pl.BlockSpec((Dd, CB), lambda c: (0, c)),
        ],
        out_specs=pl.BlockSpec((CB // 2, 2 * Dd), lambda c: (c, 0)),
        out_shape=jax.ShapeDtypeStruct((V // 2, 2 * Dd), jnp.float32),
    )(perm_c, ET)


def _lstm_tc(x2, par, Wk, Wr, b2):
    T, B, D2 = x2.shape
    D = D2 // 2
    H4 = Wk.shape[1]
    H = H4 // 4

    def body(x_ref, p_ref, wk_ref, wr_ref, b_ref, y_ref, hl_ref, h_s, c_s):
        t = pl.program_id(0)

        @pl.when(t == 0)
        def _():
            h_s[:] = jnp.zeros_like(h_s)
            c_s[:] = jnp.zeros_like(c_s)

        xt2 = x_ref[0]
        p = p_ref[0, 0][:, None]
        xt = xt2[:, :D] + (xt2[:, D:] - xt2[:, :D]) * p
        z = (
            jnp.dot(xt, wk_ref[:], preferred_element_type=jnp.float32)
            + jnp.dot(h_s[:], wr_ref[:], preferred_element_type=jnp.float32)
            + b_ref[:]
        )
        i = jax.nn.sigmoid(z[:, :H])
        f = jax.nn.sigmoid(z[:, H : 2 * H])
        g = jnp.tanh(z[:, 2 * H : 3 * H])
        o = jax.nn.sigmoid(z[:, 3 * H :])
        c_new = f * c_s[:] + i * g
        h_new = o * jnp.tanh(c_new)
        c_s[:] = c_new
        h_s[:] = h_new
        y_ref[0] = h_new

        @pl.when(t == T - 1)
        def _():
            hl_ref[:] = h_new

    return pl.pallas_call(
        body,
        grid=(T,),
        in_specs=[
            pl.BlockSpec((1, B, D2), lambda t: (t, 0, 0)),
            pl.BlockSpec((1, 1, B), lambda t: (t, 0, 0)),
            pl.BlockSpec((D, H4), lambda t: (0, 0)),
            pl.BlockSpec((H, H4), lambda t: (0, 0)),
            pl.BlockSpec((1, H4), lambda t: (0, 0)),
        ],
        out_specs=[
            pl.BlockSpec((1, B, H), lambda t: (t, 0, 0)),
            pl.BlockSpec((B, H), lambda t: (0, 0)),
        ],
        out_shape=[
            jax.ShapeDtypeStruct((T, B, H), jnp.float32),
            jax.ShapeDtypeStruct((B, H), jnp.float32),
        ],
        scratch_shapes=[
            pltpu.VMEM((B, H), jnp.float32),
            pltpu.VMEM((B, H), jnp.float32),
        ],
    )(x2, par, Wk, Wr, b2)


def kernel(inputs, E, Wk, Wr, b):
    B, T = inputs.shape
    V, D = E.shape
    idx = jnp.transpose(inputs).reshape(-1).astype(jnp.int32)
    table2 = _repack_tc(jnp.transpose(E))
    x2_flat = _make_sc_gather(V // 2, 2 * D, T * B)(table2, idx // 2)
    x2 = x2_flat.reshape(T, B, 2 * D)
    par = (idx % 2).astype(jnp.float32).reshape(T, 1, B)
    y3, h_last = _lstm_tc(x2, par, Wk, Wr, b.reshape(1, -1))
    return (jnp.transpose(y3, (1, 0, 2)), h_last)


# speedup vs baseline: 1.2556x; 1.2556x over previous
import functools

import jax
import jax.numpy as jnp
from jax import lax
from jax.experimental import pallas as pl
from jax.experimental.pallas import tpu as pltpu
from jax.experimental.pallas import tpu_sc as plsc


def _make_sc_gather(Vp, D2, N):
    info = plsc.get_sparse_core_info()
    NC, NS, L = info.num_cores, info.num_subcores, info.num_lanes
    NW = NC * NS
    assert N % NW == 0
    per_w = N // NW
    CH = 128
    sizes = [CH] * (per_w // CH)
    if per_w % CH:
        sizes.append(per_w % CH)
    mesh = plsc.VectorSubcoreMesh(core_axis_name="c", subcore_axis_name="s")

    @functools.partial(
        pl.kernel,
        out_type=jax.ShapeDtypeStruct((N, D2), jnp.float32),
        mesh=mesh,
        scratch_types=[
            pltpu.VMEM((per_w,), jnp.int32),
            pltpu.VMEM((CH, D2), jnp.float32),
            pltpu.VMEM((CH, D2), jnp.float32),
            pltpu.SemaphoreType.DMA,
        ],
    )
    def gather_k(table_hbm, idx_hbm, out_hbm, idx_v, buf_a, buf_b, sem):
        wid = lax.axis_index("s") * NC + lax.axis_index("c")
        base = pl.multiple_of(wid * per_w, per_w)
        pltpu.sync_copy(idx_hbm.at[pl.ds(base, per_w)], idx_v)
        bufs = [buf_a, buf_b]

        def fire(g):
            sz = sizes[g]
            return pltpu.async_copy(
                table_hbm.at[idx_v.at[pl.ds(g * CH, sz)]],
                bufs[g % 2].at[pl.ds(0, sz)],
                sem,
            )

        cps = [None] * len(sizes)
        cps[0] = fire(0)
        for g in range(len(sizes)):
            if g + 1 < len(sizes):
                cps[g + 1] = fire(g + 1)
            cps[g].wait()
            sz = sizes[g]
            dst = pl.multiple_of(base + g * CH, 8)
            pltpu.sync_copy(
                bufs[g % 2].at[pl.ds(0, sz)], out_hbm.at[pl.ds(dst, sz)]
            )

    return gather_k


def _repack_tc(ET):
    Dd, V = ET.shape
    SB = 256
    NS = 32
    CB = SB * NS

    def body(perm_ref, in_ref, out_ref):
        perm = perm_ref[...]
        dn = (((1,), (1,)), ((), ()))
        for j in range(0, NS, 4):
            xb = jnp.concatenate(
                [in_ref[:, (j + k) * SB : (j + k + 1) * SB] for k in range(4)],
                axis=0,
            )
            m = lax.dot_general(perm, xb, dn,
                                preferred_element_type=jnp.float32)
            for k in range(4):
                out_ref[pl.ds((j + k) * (SB // 2), SB // 2), :] = (
                    jnp.concatenate(
                        [
                            m[: SB // 2, k * Dd : (k + 1) * Dd],
                            m[SB // 2 :, k * Dd : (k + 1) * Dd],
                        ],
                        axis=1,
                    )
                )

    ii = lax.broadcasted_iota(jnp.int32, (SB, SB), 0)
    jj = lax.broadcasted_iota(jnp.int32, (SB, SB), 1)
    perm_c = (jj == 2 * (ii & 127) + (ii >> 7)).astype(jnp.float32)

    return pl.pallas_call(
        body,
        grid=(pl.cdiv(V, CB),),
        in_specs=[
            pl.BlockSpec((SB, SB), lambda c: (0, 0)),
            pl.BlockSpec((Dd, CB), lambda c: (0, c)),
        ],
        out_specs=pl.BlockSpec((CB // 2, 2 * Dd), lambda c: (c, 0)),
        out_shape=jax.ShapeDtypeStruct((V // 2, 2 * Dd), jnp.float32),
    )(perm_c, ET)


def _lstm_tc(x2, par, Wk, Wr, b2):
    T, B, D2 = x2.shape
    D = D2 // 2
    H4 = Wk.shape[1]
    H = H4 // 4

    def body(x_ref, p_ref, wk_ref, wr_ref, b_ref, y_ref, hl_ref, h_s, c_s):
        t = pl.program_id(0)

        @pl.when(t == 0)
        def _():
            h_s[:] = jnp.zeros_like(h_s)
            c_s[:] = jnp.zeros_like(c_s)

        xt2 = x_ref[0]
        p = p_ref[0, 0][:, None]
        xt = xt2[:, :D] + (xt2[:, D:] - xt2[:, :D]) * p
        z = (
            jnp.dot(xt, wk_ref[:], preferred_element_type=jnp.float32)
            + jnp.dot(h_s[:], wr_ref[:], preferred_element_type=jnp.float32)
            + b_ref[:]
        )
        i = jax.nn.sigmoid(z[:, :H])
        f = jax.nn.sigmoid(z[:, H : 2 * H])
        g = jnp.tanh(z[:, 2 * H : 3 * H])
        o = jax.nn.sigmoid(z[:, 3 * H :])
        c_new = f * c_s[:] + i * g
        h_new = o * jnp.tanh(c_new)
        c_s[:] = c_new
        h_s[:] = h_new
        y_ref[0] = h_new

        @pl.when(t == T - 1)
        def _():
            hl_ref[:] = h_new

    return pl.pallas_call(
        body,
        grid=(T,),
        in_specs=[
            pl.BlockSpec((1, B, D2), lambda t: (t, 0, 0)),
            pl.BlockSpec((1, 1, B), lambda t: (t, 0, 0)),
            pl.BlockSpec((D, H4), lambda t: (0, 0)),
            pl.BlockSpec((H, H4), lambda t: (0, 0)),
            pl.BlockSpec((1, H4), lambda t: (0, 0)),
        ],
        out_specs=[
            pl.BlockSpec((1, B, H), lambda t: (t, 0, 0)),
            pl.BlockSpec((B, H), lambda t: (0, 0)),
        ],
        out_shape=[
            jax.ShapeDtypeStruct((T, B, H), jnp.float32),
            jax.ShapeDtypeStruct((B, H), jnp.float32),
        ],
        scratch_shapes=[
            pltpu.VMEM((B, H), jnp.float32),
            pltpu.VMEM((B, H), jnp.float32),
        ],
    )(x2, par, Wk, Wr, b2)


def kernel(inputs, E, Wk, Wr, b):
    B, T = inputs.shape
    V, D = E.shape
    idx = jnp.transpose(inputs).reshape(-1).astype(jnp.int32)
    table2 = _repack_tc(jnp.transpose(E))
    x2_flat = _make_sc_gather(V // 2, 2 * D, T * B)(table2, idx // 2)
    x2 = x2_flat.reshape(T, B, 2 * D)
    par = (idx % 2).astype(jnp.float32).reshape(T, 1, B)
    y3, h_last = _lstm_tc(x2, par, Wk, Wr, b.reshape(1, -1))
    return (jnp.transpose(y3, (1, 0, 2)), h_last)


# speedup vs baseline: 1.4467x; 1.1522x over previous
import functools

import jax
import jax.numpy as jnp
from jax import lax
from jax.experimental import pallas as pl
from jax.experimental.pallas import tpu as pltpu
from jax.experimental.pallas import tpu_sc as plsc


def _make_sc_gather(Vp, D2, N):
    info = plsc.get_sparse_core_info()
    NC, NS, L = info.num_cores, info.num_subcores, info.num_lanes
    NW = NC * NS
    assert N % NW == 0
    per_w = N // NW
    CH = 128
    sizes = [CH] * (per_w // CH)
    if per_w % CH:
        sizes.append(per_w % CH)
    mesh = plsc.VectorSubcoreMesh(core_axis_name="c", subcore_axis_name="s")

    @functools.partial(
        pl.kernel,
        out_type=jax.ShapeDtypeStruct((N, D2), jnp.float32),
        mesh=mesh,
        scratch_types=[
            pltpu.VMEM((per_w,), jnp.int32),
            pltpu.VMEM((CH, D2), jnp.float32),
            pltpu.VMEM((CH, D2), jnp.float32),
            pltpu.SemaphoreType.DMA,
        ],
    )
    def gather_k(table_hbm, idx_hbm, out_hbm, idx_v, buf_a, buf_b, sem):
        wid = lax.axis_index("s") * NC + lax.axis_index("c")
        base = pl.multiple_of(wid * per_w, per_w)
        pltpu.sync_copy(idx_hbm.at[pl.ds(base, per_w)], idx_v)
        bufs = [buf_a, buf_b]

        def fire(g):
            sz = sizes[g]
            return pltpu.async_copy(
                table_hbm.at[idx_v.at[pl.ds(g * CH, sz)]],
                bufs[g % 2].at[pl.ds(0, sz)],
                sem,
            )

        cps = [None] * len(sizes)
        cps[0] = fire(0)
        for g in range(len(sizes)):
            if g + 1 < len(sizes):
                cps[g + 1] = fire(g + 1)
            cps[g].wait()
            sz = sizes[g]
            dst = pl.multiple_of(base + g * CH, 8)
            pltpu.sync_copy(
                bufs[g % 2].at[pl.ds(0, sz)], out_hbm.at[pl.ds(dst, sz)]
            )

    return gather_k


def _repack_tc(ET):
    Dd, V = ET.shape
    SB = 256
    NS = 64
    CB = SB * NS

    def body(perm_ref, in_ref, out_ref):
        perm = perm_ref[...]
        dn = (((1,), (1,)), ((), ()))
        for j in range(0, NS, 4):
            xb = jnp.concatenate(
                [in_ref[:, (j + k) * SB : (j + k + 1) * SB] for k in range(4)],
                axis=0,
            )
            m = lax.dot_general(perm, xb, dn,
                                preferred_element_type=jnp.float32)
            for k in range(4):
                out_ref[pl.ds((j + k) * (SB // 2), SB // 2), :] = (
                    jnp.concatenate(
                        [
                            m[: SB // 2, k * Dd : (k + 1) * Dd],
                            m[SB // 2 :, k * Dd : (k + 1) * Dd],
                        ],
                        axis=1,
                    )
                )

    ii = lax.broadcasted_iota(jnp.int32, (SB, SB), 0)
    jj = lax.broadcasted_iota(jnp.int32, (SB, SB), 1)
    perm_c = (jj == 2 * (ii & 127) + (ii >> 7)).astype(jnp.float32)

    return pl.pallas_call(
        body,
        grid=(pl.cdiv(V, CB),),
        in_specs=[
            pl.BlockSpec((SB, SB), lambda c: (0, 0)),
            pl.BlockSpec((Dd, CB), lambda c: (0, c)),
        ],
        out_specs=pl.BlockSpec((CB // 2, 2 * Dd), lambda c: (c, 0)),
        out_shape=jax.ShapeDtypeStruct((V // 2, 2 * Dd), jnp.float32),
    )(perm_c, ET)


def _lstm_tc(x2, par, Wk, Wr, b2):
    T, B, D2 = x2.shape
    D = D2 // 2
    H4 = Wk.shape[1]
    H = H4 // 4

    def body(x_ref, p_ref, wk_ref, wr_ref, b_ref, y_ref, hl_ref, h_s, c_s):
        t = pl.program_id(0)

        @pl.when(t == 0)
        def _():
            h_s[:] = jnp.zeros_like(h_s)
            c_s[:] = jnp.zeros_like(c_s)

        xt2 = x_ref[0]
        p = p_ref[0, 0][:, None]
        xt = xt2[:, :D] + (xt2[:, D:] - xt2[:, :D]) * p
        z = (
            jnp.dot(xt, wk_ref[:], preferred_element_type=jnp.float32)
            + jnp.dot(h_s[:], wr_ref[:], preferred_element_type=jnp.float32)
            + b_ref[:]
        )
        i = jax.nn.sigmoid(z[:, :H])
        f = jax.nn.sigmoid(z[:, H : 2 * H])
        g = jnp.tanh(z[:, 2 * H : 3 * H])
        o = jax.nn.sigmoid(z[:, 3 * H :])
        c_new = f * c_s[:] + i * g
        h_new = o * jnp.tanh(c_new)
        c_s[:] = c_new
        h_s[:] = h_new
        y_ref[0] = h_new

        @pl.when(t == T - 1)
        def _():
            hl_ref[:] = h_new

    return pl.pallas_call(
        body,
        grid=(T,),
        in_specs=[
            pl.BlockSpec((1, B, D2), lambda t: (t, 0, 0)),
            pl.BlockSpec((1, 1, B), lambda t: (t, 0, 0)),
            pl.BlockSpec((D, H4), lambda t: (0, 0)),
            pl.BlockSpec((H, H4), lambda t: (0, 0)),
            pl.BlockSpec((1, H4), lambda t: (0, 0)),
        ],
        out_specs=[
            pl.BlockSpec((1, B, H), lambda t: (t, 0, 0)),
            pl.BlockSpec((B, H), lambda t: (0, 0)),
        ],
        out_shape=[
            jax.ShapeDtypeStruct((T, B, H), jnp.float32),
            jax.ShapeDtypeStruct((B, H), jnp.float32),
        ],
        scratch_shapes=[
            pltpu.VMEM((B, H), jnp.float32),
            pltpu.VMEM((B, H), jnp.float32),
        ],
    )(x2, par, Wk, Wr, b2)


def kernel(inputs, E, Wk, Wr, b):
    B, T = inputs.shape
    V, D = E.shape
    idx = jnp.transpose(inputs).reshape(-1).astype(jnp.int32)
    table2 = _repack_tc(jnp.transpose(E))
    x2_flat = _make_sc_gather(V // 2, 2 * D, T * B)(table2, idx // 2)
    x2 = x2_flat.reshape(T, B, 2 * D)
    par = (idx % 2).astype(jnp.float32).reshape(T, 1, B)
    y3, h_last = _lstm_tc(x2, par, Wk, Wr, b.reshape(1, -1))
    return (jnp.transpose(y3, (1, 0, 2)), h_last)


# speedup vs baseline: 1.4809x; 1.0236x over previous
import functools

import jax
import jax.numpy as jnp
from jax import lax
from jax.experimental import pallas as pl
from jax.experimental.pallas import tpu as pltpu
from jax.experimental.pallas import tpu_sc as plsc


def _make_sc_gather(Vp, D2, N):
    info = plsc.get_sparse_core_info()
    NC, NS, L = info.num_cores, info.num_subcores, info.num_lanes
    NW = NC * NS
    assert N % NW == 0
    per_w = N // NW
    CH = 128
    sizes = [CH] * (per_w // CH)
    if per_w % CH:
        sizes.append(per_w % CH)
    mesh = plsc.VectorSubcoreMesh(core_axis_name="c", subcore_axis_name="s")

    @functools.partial(
        pl.kernel,
        out_type=jax.ShapeDtypeStruct((N, D2), jnp.float32),
        mesh=mesh,
        scratch_types=[
            pltpu.VMEM((per_w,), jnp.int32),
            pltpu.VMEM((CH, D2), jnp.float32),
            pltpu.VMEM((CH, D2), jnp.float32),
            pltpu.SemaphoreType.DMA,
        ],
    )
    def gather_k(table_hbm, idx_hbm, out_hbm, idx_v, buf_a, buf_b, sem):
        wid = lax.axis_index("s") * NC + lax.axis_index("c")
        base = pl.multiple_of(wid * per_w, per_w)
        pltpu.sync_copy(idx_hbm.at[pl.ds(base, per_w)], idx_v)
        bufs = [buf_a, buf_b]

        def fire(g):
            sz = sizes[g]
            return pltpu.async_copy(
                table_hbm.at[idx_v.at[pl.ds(g * CH, sz)]],
                bufs[g % 2].at[pl.ds(0, sz)],
                sem,
            )

        cps = [None] * len(sizes)
        cps[0] = fire(0)
        for g in range(len(sizes)):
            if g + 1 < len(sizes):
                cps[g + 1] = fire(g + 1)
            cps[g].wait()
            sz = sizes[g]
            dst = pl.multiple_of(base + g * CH, 8)
            pltpu.sync_copy(
                bufs[g % 2].at[pl.ds(0, sz)], out_hbm.at[pl.ds(dst, sz)]
            )

    return gather_k


def _repack_tc(ET):
    Dd, V = ET.shape
    SB = 256
    NS = 128
    CB = SB * NS

    def body(perm_ref, in_ref, out_ref):
        perm = perm_ref[...]
        dn = (((1,), (1,)), ((), ()))
        for j in range(0, NS, 4):
            xb = jnp.concatenate(
                [in_ref[:, (j + k) * SB : (j + k + 1) * SB] for k in range(4)],
                axis=0,
            )
            m = lax.dot_general(perm, xb, dn,
                                preferred_element_type=jnp.float32)
            for k in range(4):
                out_ref[pl.ds((j + k) * (SB // 2), SB // 2), :] = (
                    jnp.concatenate(
                        [
                            m[: SB // 2, k * Dd : (k + 1) * Dd],
                            m[SB // 2 :, k * Dd : (k + 1) * Dd],
                        ],
                        axis=1,
                    )
                )

    ii = lax.broadcasted_iota(jnp.int32, (SB, SB), 0)
    jj = lax.broadcasted_iota(jnp.int32, (SB, SB), 1)
    perm_c = (jj == 2 * (ii & 127) + (ii >> 7)).astype(jnp.float32)

    return pl.pallas_call(
        body,
        grid=(pl.cdiv(V, CB),),
        in_specs=[
            pl.BlockSpec((SB, SB), lambda c: (0, 0)),
            pl.BlockSpec((Dd, CB), lambda c: (0, c)),
        ],
        out_specs=pl.BlockSpec((CB // 2, 2 * Dd), lambda c: (c, 0)),
        out_shape=jax.ShapeDtypeStruct((V // 2, 2 * Dd), jnp.float32),
    )(perm_c, ET)


def _lstm_tc(x2, par, Wk, Wr, b2):
    T, B, D2 = x2.shape
    D = D2 // 2
    H4 = Wk.shape[1]
    H = H4 // 4

    def body(x_ref, p_ref, wk_ref, wr_ref, b_ref, y_ref, hl_ref, h_s, c_s):
        t = pl.program_id(0)

        @pl.when(t == 0)
        def _():
            h_s[:] = jnp.zeros_like(h_s)
            c_s[:] = jnp.zeros_like(c_s)

        xt2 = x_ref[0]
        p = p_ref[0, 0][:, None]
        xt = xt2[:, :D] + (xt2[:, D:] - xt2[:, :D]) * p
        z = (
            jnp.dot(xt, wk_ref[:], preferred_element_type=jnp.float32)
            + jnp.dot(h_s[:], wr_ref[:], preferred_element_type=jnp.float32)
            + b_ref[:]
        )
        i = jax.nn.sigmoid(z[:, :H])
        f = jax.nn.sigmoid(z[:, H : 2 * H])
        g = jnp.tanh(z[:, 2 * H : 3 * H])
        o = jax.nn.sigmoid(z[:, 3 * H :])
        c_new = f * c_s[:] + i * g
        h_new = o * jnp.tanh(c_new)
        c_s[:] = c_new
        h_s[:] = h_new
        y_ref[0] = h_new

        @pl.when(t == T - 1)
        def _():
            hl_ref[:] = h_new

    return pl.pallas_call(
        body,
        grid=(T,),
        in_specs=[
            pl.BlockSpec((1, B, D2), lambda t: (t, 0, 0)),
            pl.BlockSpec((1, 1, B), lambda t: (t, 0, 0)),
            pl.BlockSpec((D, H4), lambda t: (0, 0)),
            pl.BlockSpec((H, H4), lambda t: (0, 0)),
            pl.BlockSpec((1, H4), lambda t: (0, 0)),
        ],
        out_specs=[
            pl.BlockSpec((1, B, H), lambda t: (t, 0, 0)),
            pl.BlockSpec((B, H), lambda t: (0, 0)),
        ],
        out_shape=[
            jax.ShapeDtypeStruct((T, B, H), jnp.float32),
            jax.ShapeDtypeStruct((B, H), jnp.float32),
        ],
        scratch_shapes=[
            pltpu.VMEM((B, H), jnp.float32),
            pltpu.VMEM((B, H), jnp.float32),
        ],
    )(x2, par, Wk, Wr, b2)


def kernel(inputs, E, Wk, Wr, b):
    B, T = inputs.shape
    V, D = E.shape
    idx = jnp.transpose(inputs).reshape(-1).astype(jnp.int32)
    table2 = _repack_tc(jnp.transpose(E))
    x2_flat = _make_sc_gather(V // 2, 2 * D, T * B)(table2, idx // 2)
    x2 = x2_flat.reshape(T, B, 2 * D)
    par = (idx % 2).astype(jnp.float32).reshape(T, 1, B)
    y3, h_last = _lstm_tc(x2, par, Wk, Wr, b.reshape(1, -1))
    return (jnp.transpose(y3, (1, 0, 2)), h_last)


# speedup vs baseline: 1.5434x; 1.0422x over previous
import functools

import jax
import jax.numpy as jnp
from jax import lax
from jax.experimental import pallas as pl
from jax.experimental.pallas import tpu as pltpu
from jax.experimental.pallas import tpu_sc as plsc


def _make_sc_gather(Vp, D2, N):
    info = plsc.get_sparse_core_info()
    NC, NS, L = info.num_cores, info.num_subcores, info.num_lanes
    NW = NC * NS
    assert N % NW == 0
    per_w = N // NW
    CH = 128
    sizes = [CH] * (per_w // CH)
    if per_w % CH:
        sizes.append(per_w % CH)
    mesh = plsc.VectorSubcoreMesh(core_axis_name="c", subcore_axis_name="s")

    @functools.partial(
        pl.kernel,
        out_type=jax.ShapeDtypeStruct((N, D2), jnp.float32),
        mesh=mesh,
        scratch_types=[
            pltpu.VMEM((per_w,), jnp.int32),
            pltpu.VMEM((CH, D2), jnp.float32),
            pltpu.VMEM((CH, D2), jnp.float32),
            pltpu.SemaphoreType.DMA,
        ],
    )
    def gather_k(table_hbm, idx_hbm, out_hbm, idx_v, buf_a, buf_b, sem):
        wid = lax.axis_index("s") * NC + lax.axis_index("c")
        base = pl.multiple_of(wid * per_w, per_w)
        pltpu.sync_copy(idx_hbm.at[pl.ds(base, per_w)], idx_v)
        bufs = [buf_a, buf_b]

        def fire(g):
            sz = sizes[g]
            return pltpu.async_copy(
                table_hbm.at[idx_v.at[pl.ds(g * CH, sz)]],
                bufs[g % 2].at[pl.ds(0, sz)],
                sem,
            )

        cps = [None] * len(sizes)
        cps[0] = fire(0)
        for g in range(len(sizes)):
            if g + 1 < len(sizes):
                cps[g + 1] = fire(g + 1)
            cps[g].wait()
            sz = sizes[g]
            dst = pl.multiple_of(base + g * CH, 8)
            pltpu.sync_copy(
                bufs[g % 2].at[pl.ds(0, sz)], out_hbm.at[pl.ds(dst, sz)]
            )

    return gather_k


def _repack_tc(ET):
    Dd, V = ET.shape
    SB = 256
    NS = 128
    CB = SB * NS

    def body(perm_ref, in_ref, out_ref):
        perm = perm_ref[...]
        dn = (((1,), (1,)), ((), ()))
        for j in range(0, NS, 4):
            xb = jnp.concatenate(
                [in_ref[:, (j + k) * SB : (j + k + 1) * SB] for k in range(4)],
                axis=0,
            )
            m = lax.dot_general(perm, xb, dn,
                                preferred_element_type=jnp.float32)
            for k in range(4):
                out_ref[pl.ds((j + k) * (SB // 2), SB // 2), :] = (
                    jnp.concatenate(
                        [
                            m[: SB // 2, k * Dd : (k + 1) * Dd],
                            m[SB // 2 :, k * Dd : (k + 1) * Dd],
                        ],
                        axis=1,
                    )
                )

    ii = lax.broadcasted_iota(jnp.int32, (SB, SB), 0)
    jj = lax.broadcasted_iota(jnp.int32, (SB, SB), 1)
    perm_c = (jj == 2 * (ii & 127) + (ii >> 7)).astype(jnp.float32)

    return pl.pallas_call(
        body,
        grid=(pl.cdiv(V, CB),),
        in_specs=[
            pl.BlockSpec((SB, SB), lambda c: (0, 0)),
            pl.BlockSpec((Dd, CB), lambda c: (0, c)),
        ],
        out_specs=pl.BlockSpec((CB // 2, 2 * Dd), lambda c: (c, 0)),
        out_shape=jax.ShapeDtypeStruct((V // 2, 2 * Dd), jnp.float32),
    )(perm_c, ET)


def _lstm_tc(x2, par, Wk, Wr, b2):
    T, B, D2 = x2.shape
    D = D2 // 2
    H4 = Wk.shape[1]
    H = H4 // 4

    def body(x_ref, p_ref, wk_ref, wr_ref, b_ref, y_ref, hl_ref, h_s, c_s):
        t = pl.program_id(0)

        @pl.when(t == 0)
        def _():
            h_s[:] = jnp.zeros_like(h_s)
            c_s[:] = jnp.zeros_like(c_s)

        def step(xt2, p, h, c):
            xt = xt2[:, :D] + (xt2[:, D:] - xt2[:, :D]) * p[:, None]
            z = (
                jnp.dot(xt, wk_ref[:], preferred_element_type=jnp.float32)
                + jnp.dot(h, wr_ref[:], preferred_element_type=jnp.float32)
                + b_ref[:]
            )
            i = jax.nn.sigmoid(z[:, :H])
            f = jax.nn.sigmoid(z[:, H : 2 * H])
            g = jnp.tanh(z[:, 2 * H : 3 * H])
            o = jax.nn.sigmoid(z[:, 3 * H :])
            c_new = f * c + i * g
            h_new = o * jnp.tanh(c_new)
            return h_new, c_new

        h0, c0 = step(x_ref[0], p_ref[0, 0], h_s[:], c_s[:])
        y_ref[0] = h0
        h1, c1 = step(x_ref[1], p_ref[1, 0], h0, c0)
        y_ref[1] = h1
        h_s[:] = h1
        c_s[:] = c1

        @pl.when(t == T // 2 - 1)
        def _():
            hl_ref[:] = h1

    return pl.pallas_call(
        body,
        grid=(T // 2,),
        in_specs=[
            pl.BlockSpec((2, B, D2), lambda t: (t, 0, 0)),
            pl.BlockSpec((2, 1, B), lambda t: (t, 0, 0)),
            pl.BlockSpec((D, H4), lambda t: (0, 0)),
            pl.BlockSpec((H, H4), lambda t: (0, 0)),
            pl.BlockSpec((1, H4), lambda t: (0, 0)),
        ],
        out_specs=[
            pl.BlockSpec((2, B, H), lambda t: (t, 0, 0)),
            pl.BlockSpec((B, H), lambda t: (0, 0)),
        ],
        out_shape=[
            jax.ShapeDtypeStruct((T, B, H), jnp.float32),
            jax.ShapeDtypeStruct((B, H), jnp.float32),
        ],
        scratch_shapes=[
            pltpu.VMEM((B, H), jnp.float32),
            pltpu.VMEM((B, H), jnp.float32),
        ],
    )(x2, par, Wk, Wr, b2)


def kernel(inputs, E, Wk, Wr, b):
    B, T = inputs.shape
    V, D = E.shape
    idx = jnp.transpose(inputs).reshape(-1).astype(jnp.int32)
    table2 = _repack_tc(jnp.transpose(E))
    x2_flat = _make_sc_gather(V // 2, 2 * D, T * B)(table2, idx // 2)
    x2 = x2_flat.reshape(T, B, 2 * D)
    par = (idx % 2).astype(jnp.float32).reshape(T, 1, B)
    y3, h_last = _lstm_tc(x2, par, Wk, Wr, b.reshape(1, -1))
    return (jnp.transpose(y3, (1, 0, 2)), h_last)
